# baseline (speedup 1.0000x reference)
import functools

import jax
import jax.numpy as jnp
from jax import lax
from jax.experimental import pallas as pl
from jax.experimental.pallas import tpu as pltpu
from jax.experimental.pallas import tpu_sc as plsc

_LANES = 16


def _run(x, tbl, n_workers, rows_per_worker, n_params):
    mesh = plsc.VectorSubcoreMesh(core_axis_name="c", subcore_axis_name="s")

    @functools.partial(
        pl.kernel,
        mesh=mesh,
        out_type=jax.ShapeDtypeStruct(x.shape, jnp.float32),
        scratch_types=[
            pltpu.VMEM((rows_per_worker, n_params), jnp.float32),
            pltpu.VMEM((32,), jnp.float32),
        ],
        compiler_params=pltpu.CompilerParams(needs_layout_passes=False),
    )
    def body(x_hbm, tbl_hbm, out_hbm, buf, tbl_v):
        wid = lax.axis_index("s") * 2 + lax.axis_index("c")
        base = wid * rows_per_worker
        pltpu.sync_copy(tbl_hbm, tbl_v)
        pltpu.sync_copy(x_hbm.at[pl.ds(base, rows_per_worker)], buf)

        lanes = lax.iota(jnp.int32, _LANES)
        off1 = jnp.where(lanes >= 14, 8, 0)
        cat3 = lanes < 2

        def row(r, carry):
            v0 = buf[r, pl.ds(0, _LANES)]
            buf[r, pl.ds(0, _LANES)] = plsc.load_gather(
                tbl_v, [v0.astype(jnp.int32)])
            v1 = buf[r, pl.ds(16, _LANES)]
            buf[r, pl.ds(16, _LANES)] = plsc.load_gather(
                tbl_v, [v1.astype(jnp.int32) + off1])
            v2 = buf[r, pl.ds(32, _LANES)]
            buf[r, pl.ds(32, _LANES)] = plsc.load_gather(
                tbl_v, [v2.astype(jnp.int32) + 8])
            v3 = buf[r, pl.ds(48, _LANES)]
            g3 = plsc.load_gather(tbl_v, [v3.astype(jnp.int32) + 8])
            buf[r, pl.ds(48, _LANES)] = jnp.where(cat3, g3, v3)
            return carry

        lax.fori_loop(0, rows_per_worker, row, 0, unroll=4)
        pltpu.sync_copy(buf, out_hbm.at[pl.ds(base, rows_per_worker)])

    return body(x, tbl)


def kernel(x, cat_values_0, cat_values_1, cat_idx_0, cat_idx_1):
    batch, n_params = x.shape
    n_workers = 32
    rows_per_worker = batch // n_workers
    tbl = jnp.zeros((32,), jnp.float32)
    tbl = tbl.at[:8].set(cat_values_0).at[8:24].set(cat_values_1)
    return _run(x, tbl, n_workers, rows_per_worker, n_params)


# speedup vs baseline: 1.5305x; 1.5305x over previous
import functools

import jax
import jax.numpy as jnp
from jax import lax
from jax.experimental import pallas as pl
from jax.experimental.pallas import tpu as pltpu
from jax.experimental.pallas import tpu_sc as plsc

_LANES = 16
_N_WORKERS = 32
_CHUNK_COLS = 128
_CHUNKS = 4


def _lookup_rows(buf, tv, r_lo, r_hi):
    def body(r, carry):
        for s in range(_CHUNK_COLS // _LANES):
            sl = pl.ds(s * _LANES, _LANES)
            v = buf[r, sl]
            buf[r, sl] = plsc.load_gather(tv, [v.astype(jnp.int32)])
        return carry

    lax.fori_loop(r_lo, r_hi, body, 0)


def _run(xt, tbl0, tbl1, n_cat0, n_cat1):
    n_params, batch = xt.shape
    stripe = batch // _N_WORKERS
    mesh = plsc.VectorSubcoreMesh(core_axis_name="c", subcore_axis_name="s")

    @functools.partial(
        pl.kernel,
        mesh=mesh,
        out_type=jax.ShapeDtypeStruct(xt.shape, jnp.float32),
        scratch_types=[
            pltpu.VMEM((n_params, _CHUNK_COLS), jnp.float32),
            pltpu.VMEM((n_params, _CHUNK_COLS), jnp.float32),
            pltpu.VMEM((n_params, _CHUNK_COLS), jnp.float32),
            pltpu.VMEM((n_params, _CHUNK_COLS), jnp.float32),
            pltpu.VMEM((_LANES,), jnp.float32),
            pltpu.VMEM((_LANES,), jnp.float32),
            pltpu.SemaphoreType.DMA,
            pltpu.SemaphoreType.DMA,
            pltpu.SemaphoreType.DMA,
            pltpu.SemaphoreType.DMA,
            pltpu.SemaphoreType.DMA,
            pltpu.SemaphoreType.DMA,
            pltpu.SemaphoreType.DMA,
            pltpu.SemaphoreType.DMA,
        ],
        compiler_params=pltpu.CompilerParams(needs_layout_passes=False),
    )
    def body(xt_hbm, t0_hbm, t1_hbm, out_hbm, b0, b1, b2, b3, tv0, tv1,
             si0, si1, si2, si3, so0, so1, so2, so3):
        wid = lax.axis_index("s") * 2 + lax.axis_index("c")
        base = wid * stripe
        bufs = (b0, b1, b2, b3)
        sins = (si0, si1, si2, si3)
        souts = (so0, so1, so2, so3)

        pltpu.sync_copy(t0_hbm, tv0)
        pltpu.sync_copy(t1_hbm, tv1)

        ins = [
            pltpu.async_copy(
                xt_hbm.at[:, pl.ds(base + k * _CHUNK_COLS, _CHUNK_COLS)],
                bufs[k], sins[k])
            for k in range(_CHUNKS)
        ]
        outs = []
        for k in range(_CHUNKS):
            ins[k].wait()
            _lookup_rows(bufs[k], tv0, 0, n_cat0)
            _lookup_rows(bufs[k], tv1, n_cat0, n_cat0 + n_cat1)
            outs.append(pltpu.async_copy(
                bufs[k],
                out_hbm.at[:, pl.ds(base + k * _CHUNK_COLS, _CHUNK_COLS)],
                souts[k]))
        for o in outs:
            o.wait()

    return body(xt, tbl0, tbl1)


def kernel(x, cat_values_0, cat_values_1, cat_idx_0, cat_idx_1):
    t0 = jnp.zeros((_LANES,), jnp.float32).at[:cat_values_0.shape[0]].set(
        cat_values_0)
    t1 = jnp.zeros((_LANES,), jnp.float32).at[:cat_values_1.shape[0]].set(
        cat_values_1)
    out_t = _run(x.T, t0, t1, cat_idx_0.shape[0], cat_idx_1.shape[0])
    return out_t.T


# speedup vs baseline: 1.6121x; 1.0533x over previous
import functools

import jax
import jax.numpy as jnp
from jax import lax
from jax.experimental import pallas as pl
from jax.experimental.pallas import tpu as pltpu
from jax.experimental.pallas import tpu_sc as plsc

_LANES = 16
_N_WORKERS = 32
_CHUNK_COLS = 128
_CHUNKS = 4
_PAT_TABLE = 1048
_SHIFT = 20


def _lookup_rows(buf, tb, r_lo, r_hi):
    @plsc.parallel_loop(r_lo, r_hi, unroll=2)
    def body(r):
        for s in range(_CHUNK_COLS // _LANES):
            sl = pl.ds(s * _LANES, _LANES)
            idx = lax.shift_right_logical(
                plsc.bitcast(buf[r, sl], jnp.int32), _SHIFT)
            buf[r, sl] = plsc.load_gather(tb, [idx])


def _run(xt, tbl0, tbl1, n_cat0, n_cat1, n_vals0):
    n_params, batch = xt.shape
    stripe = batch // _N_WORKERS
    mesh = plsc.VectorSubcoreMesh(core_axis_name="c", subcore_axis_name="s")

    @functools.partial(
        pl.kernel,
        mesh=mesh,
        out_type=jax.ShapeDtypeStruct(xt.shape, jnp.float32),
        scratch_types=[
            pltpu.VMEM((n_params, _CHUNK_COLS), jnp.float32),
            pltpu.VMEM((n_params, _CHUNK_COLS), jnp.float32),
            pltpu.VMEM((n_params, _CHUNK_COLS), jnp.float32),
            pltpu.VMEM((n_params, _CHUNK_COLS), jnp.float32),
            pltpu.VMEM((_LANES,), jnp.float32),
            pltpu.VMEM((_LANES,), jnp.float32),
            pltpu.VMEM((_PAT_TABLE,), jnp.float32),
            pltpu.VMEM((_PAT_TABLE,), jnp.float32),
            pltpu.SemaphoreType.DMA,
            pltpu.SemaphoreType.DMA,
            pltpu.SemaphoreType.DMA,
            pltpu.SemaphoreType.DMA,
            pltpu.SemaphoreType.DMA,
            pltpu.SemaphoreType.DMA,
            pltpu.SemaphoreType.DMA,
            pltpu.SemaphoreType.DMA,
        ],
        compiler_params=pltpu.CompilerParams(needs_layout_passes=False),
    )
    def body(xt_hbm, t0_hbm, t1_hbm, out_hbm, b0, b1, b2, b3, tv0, tv1,
             tb0, tb1, si0, si1, si2, si3, so0, so1, so2, so3):
        wid = lax.axis_index("s") * 2 + lax.axis_index("c")
        base = wid * stripe
        bufs = (b0, b1, b2, b3)
        sins = (si0, si1, si2, si3)
        souts = (so0, so1, so2, so3)

        pltpu.sync_copy(t0_hbm, tv0)
        pltpu.sync_copy(t1_hbm, tv1)

        ins = [
            pltpu.async_copy(
                xt_hbm.at[:, pl.ds(base + k * _CHUNK_COLS, _CHUNK_COLS)],
                bufs[k], sins[k])
            for k in range(_CHUNKS)
        ]

        lanes = lax.iota(jnp.int32, _LANES)
        pats = lax.shift_right_logical(
            plsc.bitcast(lanes.astype(jnp.float32), jnp.int32), _SHIFT)
        vals0 = plsc.load_gather(tv0, [jnp.minimum(lanes, n_vals0 - 1)])
        plsc.store_scatter(tb0, [pats], vals0)
        vals1 = plsc.load_gather(tv1, [lanes])
        plsc.store_scatter(tb1, [pats], vals1)

        outs = []
        for k in range(_CHUNKS):
            ins[k].wait()
            _lookup_rows(bufs[k], tb0, 0, n_cat0)
            _lookup_rows(bufs[k], tb1, n_cat0, n_cat0 + n_cat1)
            outs.append(pltpu.async_copy(
                bufs[k],
                out_hbm.at[:, pl.ds(base + k * _CHUNK_COLS, _CHUNK_COLS)],
                souts[k]))
        for o in outs:
            o.wait()

    return body(xt, tbl0, tbl1)


def kernel(x, cat_values_0, cat_values_1, cat_idx_0, cat_idx_1):
    t0 = jnp.zeros((_LANES,), jnp.float32).at[:cat_values_0.shape[0]].set(
        cat_values_0)
    t1 = jnp.zeros((_LANES,), jnp.float32).at[:cat_values_1.shape[0]].set(
        cat_values_1)
    out_t = _run(x.T, t0, t1, cat_idx_0.shape[0], cat_idx_1.shape[0],
                 cat_values_0.shape[0])
    return out_t.T
